# baseline (speedup 1.0000x reference)
import functools

import jax
import jax.numpy as jnp
from jax.experimental import pallas as pl
from jax.experimental.pallas import tpu as pltpu

B = 64
LATENT = 32
EMB = 64
KCODES = 512

_TAPS = [divmod(di + 1, 2) for di in range(4)]


def _enc_kern(xp_ref, w1_ref, b1_ref, w2_ref, b2_ref, w3_ref, b3_ref,
              o_ref, c1_ref, c2_ref, *, nb):
    rows = nb * 64

    @pl.when(pl.program_id(0) == 0)
    def _zero():
        c1_ref[...] = jnp.zeros_like(c1_ref)
        c2_ref[...] = jnp.zeros_like(c2_ref)

    xpr = xp_ref[...].reshape(nb, 8, 4, 8, 4, 16)
    for ri in range(4):
        for rj in range(4):
            y = jnp.maximum(
                jnp.dot(xpr[:, :, ri, :, rj, :].reshape(rows, 16), w1_ref[...],
                        preferred_element_type=jnp.float32) + b1_ref[...],
                0.0)
            c1_ref[ri, rj, :, 1:9, 1:9, :] = y.reshape(nb, 8, 8, 64)
    for P in range(2):
        for Q in range(2):
            acc = jnp.zeros((rows, 128), jnp.float32)
            for di in range(4):
                t = 2 * P + di - 1
                ri, oi = t % 4, (t - t % 4) // 4 + 1
                for dj in range(4):
                    u = 2 * Q + dj - 1
                    rj, oj = u % 4, (u - u % 4) // 4 + 1
                    tap = c1_ref[ri, rj, :, oi:oi + 8, oj:oj + 8, :]
                    acc += jnp.dot(tap.reshape(rows, 64), w2_ref[di * 4 + dj],
                                   preferred_element_type=jnp.float32)
            y = jnp.maximum(acc + b2_ref[...], 0.0)
            c2_ref[P, Q, :, 1:9, 1:9, :] = y.reshape(nb, 8, 8, 128)
    acc3 = jnp.zeros((rows, 256), jnp.float32)
    for di in range(4):
        a, p = _TAPS[di]
        for dj in range(4):
            b_, q = _TAPS[dj]
            tap = c2_ref[p, q, :, a:a + 8, b_:b_ + 8, :]
            acc3 += jnp.dot(tap.reshape(rows, 128), w3_ref[di * 4 + dj],
                            preferred_element_type=jnp.float32)
    o_ref[...] = jnp.maximum(acc3 + b3_ref[...], 0.0)


def _encoder(xp, w1, b1, w2, b2, w3, b3, nb):
    ng = B // nb
    return pl.pallas_call(
        functools.partial(_enc_kern, nb=nb),
        grid=(ng,),
        in_specs=[
            pl.BlockSpec((nb * 1024, 16), lambda g: (g, 0)),
            pl.BlockSpec((16, 64), lambda g: (0, 0)),
            pl.BlockSpec((1, 64), lambda g: (0, 0)),
            pl.BlockSpec((16, 64, 128), lambda g: (0, 0, 0)),
            pl.BlockSpec((1, 128), lambda g: (0, 0)),
            pl.BlockSpec((16, 128, 256), lambda g: (0, 0, 0)),
            pl.BlockSpec((1, 256), lambda g: (0, 0)),
        ],
        out_specs=pl.BlockSpec((nb * 64, 256), lambda g: (g, 0)),
        out_shape=jax.ShapeDtypeStruct((B * 64, 256), jnp.float32),
        scratch_shapes=[
            pltpu.VMEM((4, 4, nb, 10, 10, 64), jnp.float32),
            pltpu.VMEM((2, 2, nb, 10, 10, 128), jnp.float32),
        ],
        compiler_params=pltpu.CompilerParams(
            dimension_semantics=("arbitrary",)),
    )(xp, w1, b1.reshape(1, 64), w2, b2.reshape(1, 128),
      w3, b3.reshape(1, 256))


_S_MID = 8
_S_T1 = 9
_S_T23 = 25
_NSTEP = 33


def _mega_kern(h3_ref, wf1_ref, bf1_ref, wfe_ref, bfe_ref, e_ref, et_ref,
               wd1_ref, bd1_ref, kt1_ref, bt1_ref, kt2_ref, bt2_ref,
               bd3_ref, bt3_ref, o_ref, acc_ref, d_ref, t1_ref):
    s = pl.program_id(0)

    @pl.when(s == 0)
    def _init():
        acc_ref[...] = jnp.zeros_like(acc_ref)

    @pl.when(s < _S_MID)
    def _fc1():
        acc_ref[...] += jnp.dot(h3_ref[...], wf1_ref[...],
                                preferred_element_type=jnp.float32)

    @pl.when(s == _S_MID)
    def _mid():
        h1 = jnp.maximum(acc_ref[...] + bf1_ref[...], 0.0)
        pe = jnp.maximum(
            jnp.dot(h1, wfe_ref[...], preferred_element_type=jnp.float32)
            + bfe_ref[...], 0.0)
        et = et_ref[...]
        e = e_ref[...]
        en2 = 0.25 * jnp.sum(et * et, axis=0, keepdims=True)
        cols = []
        for latent in range(LATENT):
            zl = pe[:, latent * EMB:(latent + 1) * EMB]
            dist = en2 + jnp.dot(zl, et,
                                 preferred_element_type=jnp.float32)
            mn = jnp.min(dist, axis=1, keepdims=True)
            iota = jax.lax.broadcasted_iota(jnp.int32, dist.shape, 1)
            idx = jnp.min(jnp.where(dist == mn, iota, KCODES), axis=1,
                          keepdims=True)
            onehot = (iota == idx).astype(jnp.float32)
            cols.append(jnp.dot(onehot, e,
                                preferred_element_type=jnp.float32))
        fcv = jnp.concatenate(cols, axis=1)
        d_ref[...] = jnp.maximum(
            jnp.dot(fcv, wd1_ref[...], preferred_element_type=jnp.float32)
            + bd1_ref[...], 0.0)

    @pl.when((s >= _S_T1) & (s < _S_T23))
    def _t1():
        j = s - _S_T1
        d = d_ref[...]
        kt = kt1_ref[...]
        for p in range(4):
            y = jnp.dot(d, kt[p], preferred_element_type=jnp.float32)
            t1_ref[pl.ds(j * 256 + p * 64, 64), :] = jnp.maximum(
                y + bt1_ref[...], 0.0)

    @pl.when(s >= _S_T23)
    def _t23():
        mred = s - _S_T23
        a2 = t1_ref[pl.ds(mred * 512, 512), :]
        y2 = jnp.maximum(
            jnp.dot(a2, kt2_ref[...], preferred_element_type=jnp.float32)
            + bt2_ref[...], 0.0)
        o_ref[...] = jax.nn.sigmoid(
            jnp.dot(y2, bd3_ref[...], preferred_element_type=jnp.float32)
            + bt3_ref[...])


def _mega(h3, Wf1, bf1, Wfe, bfe, embeds, et, Wd1, bd1, kt1r, bt1r,
          kt2r, bt2r, bd3, bt3r):
    c = lambda s: (0, 0)
    c3 = lambda s: (0, 0, 0)
    return pl.pallas_call(
        _mega_kern,
        grid=(_NSTEP,),
        in_specs=[
            pl.BlockSpec((B, 2048), lambda s: (0, jnp.minimum(s, 7))),
            pl.BlockSpec((2048, 1024), lambda s: (jnp.minimum(s, 7), 0)),
            pl.BlockSpec((1, 1024), c),
            pl.BlockSpec((1024, 2048), c),
            pl.BlockSpec((1, 2048), c),
            pl.BlockSpec((KCODES, EMB), c),
            pl.BlockSpec((EMB, KCODES), c),
            pl.BlockSpec((2048, 1024), c),
            pl.BlockSpec((1, 1024), c),
            pl.BlockSpec((4, 1024, 256),
                         lambda s: (jnp.clip(s - _S_T1, 0, 15), 0, 0)),
            pl.BlockSpec((1, 256), c),
            pl.BlockSpec((256, 2048), c),
            pl.BlockSpec((1, 2048), c),
            pl.BlockSpec((2048, 64), c),
            pl.BlockSpec((1, 64), c),
        ],
        out_specs=pl.BlockSpec((512, 64),
                               lambda s: (jnp.clip(s - _S_T23, 0, 7), 0)),
        out_shape=jax.ShapeDtypeStruct((B * 64, 64), jnp.float32),
        scratch_shapes=[
            pltpu.VMEM((B, 1024), jnp.float32),
            pltpu.VMEM((B, 1024), jnp.float32),
            pltpu.VMEM((4096, 256), jnp.float32),
        ],
        compiler_params=pltpu.CompilerParams(
            dimension_semantics=("arbitrary",)),
    )(h3, Wf1, bf1.reshape(1, 1024), Wfe, bfe.reshape(1, 2048), embeds, et,
      Wd1, bd1.reshape(1, 1024), kt1r, bt1r, kt2r, bt2r, bd3, bt3r)


def _final_transpose(y):
    return (y.reshape(8, 8, B, 4, 4, 2, 2)
             .transpose(2, 0, 3, 5, 1, 4, 6)
             .reshape(B, 64, 64, 1))


def _im2col16(x):
    hh = 32
    xr = x.reshape(B, hh, 2, hh, 2)
    ph = [[jnp.pad(xr[:, :, p, :, q], ((0, 0), (1, 1), (1, 1)))
           for q in range(2)] for p in range(2)]
    taps = []
    for di in range(4):
        a, p = _TAPS[di]
        for dj in range(4):
            b_, q = _TAPS[dj]
            taps.append(ph[p][q][:, a:a + hh, b_:b_ + hh])
    return jnp.stack(taps, axis=-1).reshape(B * 1024, 16)


def kernel(x, Wc1, bc1, Wc2, bc2, Wc3, bc3, Wf1, bf1, Wfe, bfe, embeds,
           Wd1, bd1, Kt1, bt1, Kt2, bt2, Kt3, bt3):
    h = _encoder(_im2col16(x), Wc1.reshape(16, 64), bc1,
                 Wc2.reshape(16, 64, 128), bc2,
                 Wc3.reshape(16, 128, 256), bc3, 16)
    kt1r = Kt1.reshape(64, 1024, 256)
    bt1r = bt1.reshape(1, 256)
    kt2r = Kt2.transpose(2, 0, 1, 3).reshape(256, 2048)
    bt2r = jnp.tile(bt2, 16).reshape(1, 2048)
    bd3 = jnp.kron(jnp.eye(16, dtype=jnp.float32),
                   Kt3.transpose(2, 0, 1, 3).reshape(128, 4))
    bt3r = jnp.tile(bt3, 64).reshape(1, 64)
    y = _mega(h.reshape(B, 16384), Wf1, bf1, Wfe, bfe, embeds,
              -2.0 * embeds.T,
              Wd1, bd1, kt1r, bt1r, kt2r, bt2r, bd3, bt3r)
    return _final_transpose(y)


# speedup vs baseline: 1.2572x; 1.2572x over previous
import functools

import jax
import jax.numpy as jnp
from jax.experimental import pallas as pl
from jax.experimental.pallas import tpu as pltpu

B = 64
LATENT = 32
EMB = 64
KCODES = 512

_TAPS = [divmod(di + 1, 2) for di in range(4)]


def _enc_kern(xp_ref, w1_ref, b1_ref, w2_ref, b2_ref, w3_ref, b3_ref,
              o_ref, c1_ref, c2_ref, *, nb):
    rows = nb * 64

    @pl.when(pl.program_id(0) == 0)
    def _zero():
        c1_ref[...] = jnp.zeros_like(c1_ref)
        c2_ref[...] = jnp.zeros_like(c2_ref)

    xpr = xp_ref[...].reshape(nb, 4, 4, 8, 8, 16)
    for ri in range(4):
        for rj in range(4):
            y = jnp.maximum(
                jnp.dot(xpr[:, ri, rj].reshape(rows, 16), w1_ref[...],
                        preferred_element_type=jnp.float32) + b1_ref[...],
                0.0)
            c1_ref[ri, rj, :, 1:9, 1:9, :] = y.reshape(nb, 8, 8, 64)
    for P in range(2):
        for Q in range(2):
            acc = jnp.zeros((rows, 128), jnp.float32)
            for di in range(4):
                t = 2 * P + di - 1
                ri, oi = t % 4, (t - t % 4) // 4 + 1
                for dj in range(4):
                    u = 2 * Q + dj - 1
                    rj, oj = u % 4, (u - u % 4) // 4 + 1
                    tap = c1_ref[ri, rj, :, oi:oi + 8, oj:oj + 8, :]
                    acc += jnp.dot(tap.reshape(rows, 64), w2_ref[di * 4 + dj],
                                   preferred_element_type=jnp.float32)
            y = jnp.maximum(acc + b2_ref[...], 0.0)
            c2_ref[P, Q, :, 1:9, 1:9, :] = y.reshape(nb, 8, 8, 128)
    acc3 = jnp.zeros((rows, 256), jnp.float32)
    for di in range(4):
        a, p = _TAPS[di]
        for dj in range(4):
            b_, q = _TAPS[dj]
            tap = c2_ref[p, q, :, a:a + 8, b_:b_ + 8, :]
            acc3 += jnp.dot(tap.reshape(rows, 128), w3_ref[di * 4 + dj],
                            preferred_element_type=jnp.float32)
    o_ref[...] = jnp.maximum(acc3 + b3_ref[...], 0.0)


def _encoder(xp, w1, b1, w2, b2, w3, b3, nb):
    ng = B // nb
    return pl.pallas_call(
        functools.partial(_enc_kern, nb=nb),
        grid=(ng,),
        in_specs=[
            pl.BlockSpec((nb * 1024, 16), lambda g: (g, 0)),
            pl.BlockSpec((16, 64), lambda g: (0, 0)),
            pl.BlockSpec((1, 64), lambda g: (0, 0)),
            pl.BlockSpec((16, 64, 128), lambda g: (0, 0, 0)),
            pl.BlockSpec((1, 128), lambda g: (0, 0)),
            pl.BlockSpec((16, 128, 256), lambda g: (0, 0, 0)),
            pl.BlockSpec((1, 256), lambda g: (0, 0)),
        ],
        out_specs=pl.BlockSpec((nb * 64, 256), lambda g: (g, 0)),
        out_shape=jax.ShapeDtypeStruct((B * 64, 256), jnp.float32),
        scratch_shapes=[
            pltpu.VMEM((4, 4, nb, 10, 10, 64), jnp.float32),
            pltpu.VMEM((2, 2, nb, 10, 10, 128), jnp.float32),
        ],
        compiler_params=pltpu.CompilerParams(
            dimension_semantics=("arbitrary",)),
    )(xp, w1, b1.reshape(1, 64), w2, b2.reshape(1, 128),
      w3, b3.reshape(1, 256))


_S_MID = 8
_S_T1 = 9
_S_T23 = 25
_NSTEP = 33


def _mega_kern(h3_ref, wf1_ref, bf1_ref, wfe_ref, bfe_ref, e_ref, et_ref,
               wd1_ref, bd1_ref, kt1_ref, bt1_ref, kt2_ref, bt2_ref,
               bd3_ref, bt3_ref, o_ref, acc_ref, d_ref, t1_ref):
    s = pl.program_id(0)

    @pl.when(s == 0)
    def _init():
        acc_ref[...] = jnp.zeros_like(acc_ref)

    @pl.when(s < _S_MID)
    def _fc1():
        acc_ref[...] += jnp.dot(h3_ref[...], wf1_ref[...],
                                preferred_element_type=jnp.float32)

    @pl.when(s == _S_MID)
    def _mid():
        h1 = jnp.maximum(acc_ref[...] + bf1_ref[...], 0.0)
        pe = jnp.maximum(
            jnp.dot(h1, wfe_ref[...], preferred_element_type=jnp.float32)
            + bfe_ref[...], 0.0)
        et = et_ref[...]
        e = e_ref[...]
        en2 = 0.25 * jnp.sum(et * et, axis=0, keepdims=True)
        cols = []
        for latent in range(LATENT):
            zl = pe[:, latent * EMB:(latent + 1) * EMB]
            dist = en2 + jnp.dot(zl, et,
                                 preferred_element_type=jnp.float32)
            mn = jnp.min(dist, axis=1, keepdims=True)
            iota = jax.lax.broadcasted_iota(jnp.int32, dist.shape, 1)
            idx = jnp.min(jnp.where(dist == mn, iota, KCODES), axis=1,
                          keepdims=True)
            onehot = (iota == idx).astype(jnp.float32)
            cols.append(jnp.dot(onehot, e,
                                preferred_element_type=jnp.float32))
        fcv = jnp.concatenate(cols, axis=1)
        d_ref[...] = jnp.maximum(
            jnp.dot(fcv, wd1_ref[...], preferred_element_type=jnp.float32)
            + bd1_ref[...], 0.0)

    @pl.when((s >= _S_T1) & (s < _S_T23))
    def _t1():
        j = s - _S_T1
        d = d_ref[...]
        kt = kt1_ref[...]
        for p in range(4):
            y = jnp.dot(d, kt[p], preferred_element_type=jnp.float32)
            t1_ref[pl.ds(j * 256 + p * 64, 64), :] = jnp.maximum(
                y + bt1_ref[...], 0.0)

    @pl.when(s >= _S_T23)
    def _t23():
        mred = s - _S_T23
        a2 = t1_ref[pl.ds(mred * 512, 512), :]
        y2 = jnp.maximum(
            jnp.dot(a2, kt2_ref[...], preferred_element_type=jnp.float32)
            + bt2_ref[...], 0.0)
        o_ref[...] = jax.nn.sigmoid(
            jnp.dot(y2, bd3_ref[...], preferred_element_type=jnp.float32)
            + bt3_ref[...])


def _mega(h3, Wf1, bf1, Wfe, bfe, embeds, et, Wd1, bd1, kt1r, bt1r,
          kt2r, bt2r, bd3, bt3r):
    c = lambda s: (0, 0)
    c3 = lambda s: (0, 0, 0)
    return pl.pallas_call(
        _mega_kern,
        grid=(_NSTEP,),
        in_specs=[
            pl.BlockSpec((B, 2048), lambda s: (0, jnp.minimum(s, 7))),
            pl.BlockSpec((2048, 1024), lambda s: (jnp.minimum(s, 7), 0)),
            pl.BlockSpec((1, 1024), c),
            pl.BlockSpec((1024, 2048), c),
            pl.BlockSpec((1, 2048), c),
            pl.BlockSpec((KCODES, EMB), c),
            pl.BlockSpec((EMB, KCODES), c),
            pl.BlockSpec((2048, 1024), c),
            pl.BlockSpec((1, 1024), c),
            pl.BlockSpec((4, 1024, 256),
                         lambda s: (jnp.clip(s - _S_T1, 0, 15), 0, 0)),
            pl.BlockSpec((1, 256), c),
            pl.BlockSpec((256, 2048), c),
            pl.BlockSpec((1, 2048), c),
            pl.BlockSpec((2048, 64), c),
            pl.BlockSpec((1, 64), c),
        ],
        out_specs=pl.BlockSpec((512, 64),
                               lambda s: (jnp.clip(s - _S_T23, 0, 7), 0)),
        out_shape=jax.ShapeDtypeStruct((B * 64, 64), jnp.float32),
        scratch_shapes=[
            pltpu.VMEM((B, 1024), jnp.float32),
            pltpu.VMEM((B, 1024), jnp.float32),
            pltpu.VMEM((4096, 256), jnp.float32),
        ],
        compiler_params=pltpu.CompilerParams(
            dimension_semantics=("arbitrary",)),
    )(h3, Wf1, bf1.reshape(1, 1024), Wfe, bfe.reshape(1, 2048), embeds, et,
      Wd1, bd1.reshape(1, 1024), kt1r, bt1r, kt2r, bt2r, bd3, bt3r)


def _final_transpose(y):
    return (y.reshape(8, 8, B, 4, 4, 2, 2)
             .transpose(2, 0, 3, 5, 1, 4, 6)
             .reshape(B, 64, 64, 1))


def _im2col8(x):
    x8p = jnp.pad(x.reshape(B, 8, 8, 8, 8).transpose(2, 4, 0, 1, 3),
                  ((0, 0), (0, 0), (0, 0), (1, 1), (1, 1)))
    def off(r, d):
        t = 2 * r + d - 1
        s = t % 8
        return s, (t - s) // 8 + 1
    ris = []
    for ri in range(4):
        rjs = []
        for rj in range(4):
            taps = []
            for di in range(4):
                si, oi = off(ri, di)
                for dj in range(4):
                    sj, oj = off(rj, dj)
                    taps.append(x8p[si, sj, :, oi:oi + 8, oj:oj + 8])
            rjs.append(jnp.stack(taps, axis=-1))
        ris.append(jnp.stack(rjs, axis=1))
    return jnp.stack(ris, axis=1).reshape(B * 1024, 16)


def kernel(x, Wc1, bc1, Wc2, bc2, Wc3, bc3, Wf1, bf1, Wfe, bfe, embeds,
           Wd1, bd1, Kt1, bt1, Kt2, bt2, Kt3, bt3):
    h = _encoder(_im2col8(x), Wc1.reshape(16, 64), bc1,
                 Wc2.reshape(16, 64, 128), bc2,
                 Wc3.reshape(16, 128, 256), bc3, 16)
    kt1r = Kt1.reshape(64, 1024, 256)
    bt1r = bt1.reshape(1, 256)
    kt2r = Kt2.transpose(2, 0, 1, 3).reshape(256, 2048)
    bt2r = jnp.tile(bt2, 16).reshape(1, 2048)
    bd3 = jnp.kron(jnp.eye(16, dtype=jnp.float32),
                   Kt3.transpose(2, 0, 1, 3).reshape(128, 4))
    bt3r = jnp.tile(bt3, 64).reshape(1, 64)
    y = _mega(h.reshape(B, 16384), Wf1, bf1, Wfe, bfe, embeds,
              -2.0 * embeds.T,
              Wd1, bd1, kt1r, bt1r, kt2r, bt2r, bd3, bt3r)
    return _final_transpose(y)
